# baseline (speedup 1.0000x reference)
import functools

import jax
import jax.numpy as jnp
from jax import lax
from jax.experimental import pallas as pl
from jax.experimental.pallas import tpu as pltpu
from jax.experimental.pallas import tpu_sc as plsc

B = 64
C = 768
D = 24 * 24
ROWS = B * C

NC = 2
NS = 16
NW = NC * NS
ROWS_PER_W = ROWS // NW
BATCHES_PER_W = ROWS_PER_W // C
CHUNK = 64
NCHUNK = ROWS_PER_W // CHUNK
CHUNKS_PER_BATCH = C // CHUNK


def _sc_shuffle(x_hbm, idx_hbm, out_hbm, idx_v, gidx_v, buf0, buf1,
                sem_g0, sem_g1, sem_o0, sem_o1):
    cid = lax.axis_index("c")
    sid = lax.axis_index("s")
    wid = sid * NC + cid
    base_row = wid * ROWS_PER_W
    first_batch = wid * BATCHES_PER_W

    pltpu.sync_copy(idx_hbm, idx_v)

    for ci in range(NCHUNK):
        h = ci // CHUNKS_PER_BATCH
        j0 = (ci % CHUNKS_PER_BATCH) * CHUNK
        row_base = (first_batch + h) * C
        for ki in range(CHUNK // 16):
            vals = idx_v[pl.ds(j0 + 16 * ki, 16)] + row_base
            gidx_v[ci, pl.ds(16 * ki, 16)] = vals

    bufs = (buf0, buf1)
    gsems = (sem_g0, sem_g1)
    osems = (sem_o0, sem_o1)
    gathers = [None, None]
    outs = [None, None]

    for ci in range(2):
        gathers[ci] = pltpu.async_copy(
            x_hbm.at[gidx_v.at[ci]], bufs[ci], gsems[ci])

    for ci in range(NCHUNK):
        b = ci % 2
        gathers[b].wait()
        outs[b] = pltpu.async_copy(
            bufs[b], out_hbm.at[pl.ds(base_row + ci * CHUNK, CHUNK)],
            osems[b])
        nxt = ci + 2
        if nxt < NCHUNK:
            outs[b].wait()
            outs[b] = None
            gathers[b] = pltpu.async_copy(
                x_hbm.at[gidx_v.at[nxt]], bufs[b], gsems[b])

    for b in range(2):
        if outs[b] is not None:
            outs[b].wait()


@jax.jit
def _shuffle(x, forward_shuffle_idx):
    xr = x.reshape(ROWS, D)
    mesh = plsc.VectorSubcoreMesh(core_axis_name="c", subcore_axis_name="s")
    run = pl.kernel(
        _sc_shuffle,
        out_type=jax.ShapeDtypeStruct((ROWS, D), jnp.float32),
        mesh=mesh,
        scratch_types=[
            pltpu.VMEM((C,), jnp.int32),
            pltpu.VMEM((NCHUNK, CHUNK), jnp.int32),
            pltpu.VMEM((CHUNK, D), jnp.float32),
            pltpu.VMEM((CHUNK, D), jnp.float32),
            pltpu.SemaphoreType.DMA,
            pltpu.SemaphoreType.DMA,
            pltpu.SemaphoreType.DMA,
            pltpu.SemaphoreType.DMA,
        ],
        compiler_params=pltpu.CompilerParams(use_tc_tiling_on_sc=False),
    )
    out = run(xr, forward_shuffle_idx)
    return out.reshape(B, C, 24, 24)


def kernel(x, forward_shuffle_idx):
    return (_shuffle(x, forward_shuffle_idx), 0)
